# baseline (speedup 1.0000x reference)
import functools

import jax
import jax.numpy as jnp
from jax import lax
from jax.experimental import pallas as pl
from jax.experimental.pallas import tpu as pltpu
from jax.experimental.pallas import tpu_sc as plsc

_SQRT1_2 = 0.7071067811865476
_NT = 512


def _gelu(x):
    return 0.5 * x * (1.0 + lax.erf(x * _SQRT1_2))


def _mm(x, w):
    m, s, c = x.shape
    y = lax.dot_general(x.reshape(m * s, c), w, (((1,), (1,)), ((), ())),
                        preferred_element_type=jnp.float32)
    return y.reshape(m, s, w.shape[0])


def _s1_body(if_ref, pf_ref, w1_ref, at_ref, bvt_ref):
    ifb = if_ref[...]
    x33 = jnp.concatenate([ifb, pf_ref[...]], axis=0)
    w1 = w1_ref[...]
    at_ref[...] = lax.dot_general(ifb, w1[:, 0:32], (((0,), (1,)), ((), ())),
                                  preferred_element_type=jnp.float32)
    bvt_ref[...] = lax.dot_general(x33, w1[:, 32:65], (((0,), (1,)), ((), ())),
                                   preferred_element_type=jnp.float32)


def _stage1(if2, pf2, w1, n):
    grid = n // _NT
    return pl.pallas_call(
        _s1_body,
        grid=(grid,),
        in_specs=[
            pl.BlockSpec((32, _NT), lambda g: (0, g)),
            pl.BlockSpec((1, _NT), lambda g: (0, g)),
            pl.BlockSpec((32, 67), lambda g: (0, 0)),
        ],
        out_specs=[
            pl.BlockSpec((_NT, 32), lambda g: (g, 0)),
            pl.BlockSpec((_NT, 32), lambda g: (g, 0)),
        ],
        out_shape=[jax.ShapeDtypeStruct((n, 32), jnp.float32),
                   jax.ShapeDtypeStruct((n, 32), jnp.float32)],
    )(if2, pf2, w1)


def _sc_gather(bvt, idxf, pfflat, npts, n):
    NW = 32
    rw = npts // NW
    ch = 1008
    nch = rw // ch
    mesh = plsc.VectorSubcoreMesh(core_axis_name="c", subcore_axis_name="s")

    @functools.partial(
        pl.kernel,
        out_type=[jax.ShapeDtypeStruct((npts, 32), jnp.float32),
                  jax.ShapeDtypeStruct((npts,), jnp.float32)],
        mesh=mesh,
        scratch_types=[pltpu.VMEM((ch,), jnp.int32),
                       pltpu.VMEM((ch, 32), jnp.float32),
                       pltpu.VMEM((ch,), jnp.float32),
                       pltpu.SemaphoreType.DMA,
                       pltpu.SemaphoreType.DMA],
        compiler_params=pltpu.CompilerParams(use_tc_tiling_on_sc=False),
    )
    def k(table_hbm, idx_hbm, pf_hbm, g_out, pfg_out,
          idx_v, rows_v, pf_v, sem, sem2):
        wid = lax.axis_index("s") * 2 + lax.axis_index("c")
        base = wid * rw
        for kk in range(nch):
            cb = base + kk * ch
            pltpu.sync_copy(idx_hbm.at[pl.ds(cb, ch)], idx_v)
            cp1 = pltpu.async_copy(table_hbm.at[idx_v], rows_v, sem)
            cp2 = pltpu.async_copy(pf_hbm.at[idx_v], pf_v, sem2)
            cp1.wait()
            cp2.wait()
            pltpu.sync_copy(rows_v, g_out.at[pl.ds(cb, ch)])
            pltpu.sync_copy(pf_v, pfg_out.at[pl.ds(cb, ch)])

    return k(bvt, idxf, pfflat)


def _make_pass_body(phase):
    def body(*refs):
        g_ref, at_ref, of_ref, w1o_ref = refs[0:4]
        i = 4
        nstage = min(phase, 3)
        pairs = []
        for _ in range(nstage):
            pairs.append((refs[i], refs[i + 1]))
            i += 2
        if phase == 4:
            sc4_ref, wc_ref, bc_ref, pfg_ref = refs[i:i + 4]
            i += 4
        out_ref = refs[i]

        of = of_ref[...]
        m_, s_, _ = of.shape
        c = lax.dot_general(of.reshape(m_ * s_, 2), w1o_ref[...],
                            (((1,), (0,)), ((), ())),
                            preferred_element_type=jnp.float32)
        y = g_ref[...] + at_ref[...][None] + c.reshape(m_, s_, 32)
        xf = None
        for kk, (sc_ref, w_ref) in enumerate(pairs):
            sc = sc_ref[...]
            a = _gelu(y * sc[0][None, None, :] + sc[1][None, None, :])
            if kk == 1:
                xf = a
            y = _mm(a, w_ref[...])
        if phase < 4:
            ps = jnp.stack([jnp.sum(y, axis=(0, 1)),
                            jnp.sum(y * y, axis=(0, 1))])

            @pl.when(pl.program_id(0) == 0)
            def _():
                out_ref[...] = jnp.zeros_like(out_ref)

            out_ref[...] += ps
        else:
            sc4 = sc4_ref[...]
            xl = y * sc4[0][None, None, :] + sc4[1][None, None, :]
            xf2 = _gelu(xf + xl)
            wc = wc_ref[...]
            alpha = jnp.sum(xf2 * wc[0][None, None, :], axis=-1) + bc_ref[0]
            beta = jnp.sum(xf2 * wc[1][None, None, :], axis=-1) + bc_ref[1]
            om = jnp.sum(xf2 * wc[2][None, None, :], axis=-1) + bc_ref[2]
            om = om - jnp.max(om, axis=0, keepdims=True)
            e = jnp.exp(om)
            om = e / jnp.sum(e, axis=0, keepdims=True)
            d = jnp.sum(((alpha + 1.0) * pfg_ref[...] + beta) * om, axis=0)
            out_ref[...] = d[None]
    return body


def _run_pass(phase, g3, at, oft, w1o, extras, num, n):
    grid = n // _NT
    specs = [
        pl.BlockSpec((num, _NT, 32), lambda g: (0, g, 0)),
        pl.BlockSpec((_NT, 32), lambda g: (g, 0)),
        pl.BlockSpec((num, _NT, 2), lambda g: (0, g, 0)),
        pl.BlockSpec((2, 32), lambda g: (0, 0)),
    ]
    ops = [g3, at, oft, w1o]
    nstage = min(phase, 3)
    for kk in range(nstage):
        specs += [pl.BlockSpec((2, 32), lambda g: (0, 0)),
                  pl.BlockSpec((32, 32), lambda g: (0, 0))]
        ops += [extras[2 * kk], extras[2 * kk + 1]]
    if phase == 4:
        sc4, wc, bc, pfg3 = extras[6:]
        specs += [pl.BlockSpec((2, 32), lambda g: (0, 0)),
                  pl.BlockSpec((3, 32), lambda g: (0, 0)),
                  pl.BlockSpec(memory_space=pltpu.SMEM),
                  pl.BlockSpec((num, _NT), lambda g: (0, g))]
        ops += [sc4, wc, bc, pfg3]
        out_spec = pl.BlockSpec((1, _NT), lambda g: (0, g))
        out_shape = jax.ShapeDtypeStruct((1, n), jnp.float32)
    else:
        out_spec = pl.BlockSpec((2, 32), lambda g: (0, 0))
        out_shape = jax.ShapeDtypeStruct((2, 32), jnp.float32)
    return pl.pallas_call(
        _make_pass_body(phase),
        grid=(grid,),
        in_specs=specs,
        out_specs=out_spec,
        out_shape=out_shape,
    )(*ops)


def _scale_shift(stats, g, b, cnt):
    mean = stats[0] / cnt
    var = stats[1] / cnt - mean * mean
    s = g * lax.rsqrt(var + 1e-5)
    return jnp.stack([s, b - mean * s])


def kernel(If, Pf, Ofnum, args, W1, g1, b1, W2, g2, b2, W3, g3, b3,
           W4, g4, b4, Wc, bc):
    B_, Cfi_, H_, W_ = If.shape
    n = H_ * W_
    num = args.shape[-2]
    npts = num * n

    if2 = If.reshape(Cfi_, n)
    pf2 = Pf.reshape(1, n)
    at, bvt = _stage1(if2, pf2, W1, n)

    idxf = args.reshape(npts)
    g_rows, pfg = _sc_gather(bvt, idxf, Pf.reshape(n), npts, n)
    g3_ = g_rows.reshape(num, n, 32)
    pfg3 = pfg.reshape(num, n)

    oft = jnp.transpose(Ofnum.reshape(2, num, n), (1, 2, 0))
    w1o = jnp.transpose(W1[:, 65:67])
    cnt = jnp.float32(npts)

    st1 = _run_pass(0, g3_, at, oft, w1o, [], num, n)
    sc1 = _scale_shift(st1, g1, b1, cnt)
    st2 = _run_pass(1, g3_, at, oft, w1o, [sc1, W2], num, n)
    sc2 = _scale_shift(st2, g2, b2, cnt)
    st3 = _run_pass(2, g3_, at, oft, w1o, [sc1, W2, sc2, W3], num, n)
    sc3 = _scale_shift(st3, g3, b3, cnt)
    st4 = _run_pass(3, g3_, at, oft, w1o, [sc1, W2, sc2, W3, sc3, W4], num, n)
    sc4 = _scale_shift(st4, g4, b4, cnt)
    dout = _run_pass(4, g3_, at, oft, w1o,
                     [sc1, W2, sc2, W3, sc3, W4, sc4, Wc, bc, pfg3], num, n)
    return dout.reshape(B_, 1, H_, W_)


# speedup vs baseline: 2.0831x; 2.0831x over previous
import functools

import jax
import jax.numpy as jnp
from jax import lax
from jax.experimental import pallas as pl
from jax.experimental.pallas import tpu as pltpu
from jax.experimental.pallas import tpu_sc as plsc

_SQRT1_2 = 0.7071067811865476
_NT = 512


def _gelu(x):
    return 0.5 * x * (1.0 + lax.erf(x * _SQRT1_2))


def _mm(x, w):
    m, s, c = x.shape
    y = lax.dot_general(x.reshape(m * s, c), w, (((1,), (1,)), ((), ())),
                        preferred_element_type=jnp.float32)
    return y.reshape(m, s, w.shape[0])


def _s1_body(if_ref, pf_ref, w1_ref, at_ref, bvt_ref):
    ifb = if_ref[...]
    x33 = jnp.concatenate([ifb, pf_ref[...]], axis=0)
    w1 = w1_ref[...]
    at_ref[...] = lax.dot_general(w1[:, 0:32], ifb, (((1,), (0,)), ((), ())),
                                  preferred_element_type=jnp.float32)
    bvt_ref[...] = lax.dot_general(x33, w1[:, 32:65], (((0,), (1,)), ((), ())),
                                   preferred_element_type=jnp.float32)


def _stage1(if2, pf2, w1, n):
    grid = n // _NT
    return pl.pallas_call(
        _s1_body,
        grid=(grid,),
        in_specs=[
            pl.BlockSpec((32, _NT), lambda g: (0, g)),
            pl.BlockSpec((1, _NT), lambda g: (0, g)),
            pl.BlockSpec((32, 67), lambda g: (0, 0)),
        ],
        out_specs=[
            pl.BlockSpec((32, _NT), lambda g: (0, g)),
            pl.BlockSpec((_NT, 32), lambda g: (g, 0)),
        ],
        out_shape=[jax.ShapeDtypeStruct((32, n), jnp.float32),
                   jax.ShapeDtypeStruct((n, 32), jnp.float32)],
    )(if2, pf2, w1)


def _sc_gather(bvt, idxf, pfflat, npts, n):
    NW = 32
    rw = npts // NW
    ch = 1008
    nch = rw // ch
    mesh = plsc.VectorSubcoreMesh(core_axis_name="c", subcore_axis_name="s")

    @functools.partial(
        pl.kernel,
        out_type=[jax.ShapeDtypeStruct((npts, 32), jnp.float32),
                  jax.ShapeDtypeStruct((npts,), jnp.float32)],
        mesh=mesh,
        scratch_types=[pltpu.VMEM((ch,), jnp.int32),
                       pltpu.VMEM((ch, 32), jnp.float32),
                       pltpu.VMEM((ch,), jnp.float32),
                       pltpu.SemaphoreType.DMA,
                       pltpu.SemaphoreType.DMA],
        compiler_params=pltpu.CompilerParams(use_tc_tiling_on_sc=False),
    )
    def k(table_hbm, idx_hbm, pf_hbm, g_out, pfg_out,
          idx_v, rows_v, pf_v, sem, sem2):
        wid = lax.axis_index("s") * 2 + lax.axis_index("c")
        base = wid * rw
        for kk in range(nch):
            cb = base + kk * ch
            pltpu.sync_copy(idx_hbm.at[pl.ds(cb, ch)], idx_v)
            cp1 = pltpu.async_copy(table_hbm.at[idx_v], rows_v, sem)
            cp2 = pltpu.async_copy(pf_hbm.at[idx_v], pf_v, sem2)
            cp1.wait()
            cp2.wait()
            pltpu.sync_copy(rows_v, g_out.at[pl.ds(cb, ch)])
            pltpu.sync_copy(pf_v, pfg_out.at[pl.ds(cb, ch)])

    return k(bvt, idxf, pfflat)


def _y1_cm(g_ref, at_ref, of_ref, w1o_ref):
    g3 = g_ref[...]
    gt = jnp.transpose(g3, (0, 2, 1))
    of = of_ref[...]
    w1o = w1o_ref[...]
    at = at_ref[...]
    num = g3.shape[0]
    rows = []
    for m in range(num):
        cm = lax.dot_general(w1o, of[:, m, :], (((1,), (0,)), ((), ())),
                             preferred_element_type=jnp.float32)
        rows.append(gt[m] + at + cm)
    return jnp.stack(rows, axis=0)


def _mm_cm(x, w):
    return jnp.stack(
        [lax.dot_general(w, x[m], (((1,), (0,)), ((), ())),
                         preferred_element_type=jnp.float32)
         for m in range(x.shape[0])], axis=0)


def _acc_stats(out_ref, y):
    ps = jnp.stack([jnp.sum(y, axis=(0, 2)), jnp.sum(y * y, axis=(0, 2))])

    @pl.when(pl.program_id(0) == 0)
    def _():
        out_ref[...] = jnp.zeros_like(out_ref)

    out_ref[...] += ps


def _body0(g_ref, at_ref, of_ref, w1o_ref, st_ref):
    _acc_stats(st_ref, _y1_cm(g_ref, at_ref, of_ref, w1o_ref))


def _body1(g_ref, at_ref, of_ref, w1o_ref, sc1_ref, w2_ref, h_ref, st_ref):
    y1 = _y1_cm(g_ref, at_ref, of_ref, w1o_ref)
    sc = sc1_ref[...]
    h = _gelu(y1 * sc[0] + sc[1])
    h_ref[...] = h
    _acc_stats(st_ref, _mm_cm(h, w2_ref[...]))


def _body2(h_ref, w2f_ref, t2_ref, w3_ref, xf_ref, st_ref):
    h = h_ref[...]
    xf = _gelu(_mm_cm(h, w2f_ref[...]) + t2_ref[...][None])
    xf_ref[...] = xf
    _acc_stats(st_ref, _mm_cm(xf, w3_ref[...]))


def _body3(xf_ref, w3f_ref, t3_ref, w4_ref, st_ref):
    xf = xf_ref[...]
    l = _gelu(_mm_cm(xf, w3f_ref[...]) + t3_ref[...][None])
    _acc_stats(st_ref, _mm_cm(l, w4_ref[...]))


def _body4(xf_ref, w3f_ref, t3_ref, w4f_ref, t4_ref, wc_ref, bc_ref,
           pfg_ref, out_ref):
    xf = xf_ref[...]
    l = _gelu(_mm_cm(xf, w3f_ref[...]) + t3_ref[...][None])
    xl = _mm_cm(l, w4f_ref[...]) + t4_ref[...][None]
    xf2 = _gelu(xf + xl)
    wc = wc_ref[...]
    bcv = bc_ref[...]
    num = xf2.shape[0]
    feats = [lax.dot_general(wc, xf2[m], (((1,), (0,)), ((), ())),
                             preferred_element_type=jnp.float32) + bcv
             for m in range(num)]
    alpha = jnp.stack([f[0] for f in feats], axis=0)
    beta = jnp.stack([f[1] for f in feats], axis=0)
    om = jnp.stack([f[2] for f in feats], axis=0)
    om = om - jnp.max(om, axis=0, keepdims=True)
    e = jnp.exp(om)
    om = e / jnp.sum(e, axis=0, keepdims=True)
    d = jnp.sum(((alpha + 1.0) * pfg_ref[...] + beta) * om, axis=0)
    out_ref[...] = d[None]


_B_STAT = pl.BlockSpec((2, 32), lambda g: (0, 0))
_S_STAT = jax.ShapeDtypeStruct((2, 32), jnp.float32)


def _cm_spec(num):
    return pl.BlockSpec((num, 32, _NT), lambda g: (0, 0, g))


def _const(shape):
    nd = len(shape)
    return pl.BlockSpec(shape, lambda g: (0,) * nd)


def _scale_shift(stats, g, b, cnt):
    mean = stats[0] / cnt
    var = stats[1] / cnt - mean * mean
    s = g * lax.rsqrt(var + 1e-5)
    return s, b - mean * s


def kernel(If, Pf, Ofnum, args, W1, g1, b1, W2, g2, b2, W3, g3, b3,
           W4, g4, b4, Wc, bc):
    B_, Cfi_, H_, W_ = If.shape
    n = H_ * W_
    num = args.shape[-2]
    npts = num * n
    grid = (n // _NT,)

    if2 = If.reshape(Cfi_, n)
    pf2 = Pf.reshape(1, n)
    at, bvt = _stage1(if2, pf2, W1, n)

    idxf = args.reshape(npts)
    g_rows, pfg = _sc_gather(bvt, idxf, Pf.reshape(n), npts, n)
    g3_ = g_rows.reshape(num, n, 32)
    pfg3 = pfg.reshape(num, n)

    of3 = Ofnum.reshape(2, num, n)
    w1o = W1[:, 65:67]
    cnt = jnp.float32(npts)

    g_spec = pl.BlockSpec((num, _NT, 32), lambda g: (0, g, 0))
    at_spec = pl.BlockSpec((32, _NT), lambda g: (0, g))
    of_spec = pl.BlockSpec((2, num, _NT), lambda g: (0, 0, g))
    y1_ins = [g_spec, at_spec, of_spec, _const((32, 2))]

    st1 = pl.pallas_call(
        _body0, grid=grid, in_specs=y1_ins, out_specs=_B_STAT,
        out_shape=_S_STAT)(g3_, at, of3, w1o)
    s1, t1 = _scale_shift(st1, g1, b1, cnt)
    sc1 = jnp.stack([s1, t1]).reshape(2, 32, 1)

    h, st2 = pl.pallas_call(
        _body1, grid=grid,
        in_specs=y1_ins + [_const((2, 32, 1)), _const((32, 32))],
        out_specs=[_cm_spec(num), _B_STAT],
        out_shape=[jax.ShapeDtypeStruct((num, 32, n), jnp.float32), _S_STAT],
    )(g3_, at, of3, w1o, sc1, W2)
    s2, t2 = _scale_shift(st2, g2, b2, cnt)

    xf, st3 = pl.pallas_call(
        _body2, grid=grid,
        in_specs=[_cm_spec(num), _const((32, 32)), _const((32, 1)),
                  _const((32, 32))],
        out_specs=[_cm_spec(num), _B_STAT],
        out_shape=[jax.ShapeDtypeStruct((num, 32, n), jnp.float32), _S_STAT],
    )(h, W2 * s2[:, None], t2.reshape(32, 1), W3)
    s3, t3 = _scale_shift(st3, g3, b3, cnt)

    st4 = pl.pallas_call(
        _body3, grid=grid,
        in_specs=[_cm_spec(num), _const((32, 32)), _const((32, 1)),
                  _const((32, 32))],
        out_specs=_B_STAT, out_shape=_S_STAT,
    )(xf, W3 * s3[:, None], t3.reshape(32, 1), W4)
    s4, t4 = _scale_shift(st4, g4, b4, cnt)

    dout = pl.pallas_call(
        _body4, grid=grid,
        in_specs=[_cm_spec(num), _const((32, 32)), _const((32, 1)),
                  _const((32, 32)), _const((32, 1)), _const((3, 32)),
                  _const((3, 1)),
                  pl.BlockSpec((num, _NT), lambda g: (0, g))],
        out_specs=pl.BlockSpec((1, _NT), lambda g: (0, g)),
        out_shape=jax.ShapeDtypeStruct((1, n), jnp.float32),
    )(xf, W3 * s3[:, None], t3.reshape(32, 1), W4 * s4[:, None],
      t4.reshape(32, 1), Wc, bc.reshape(3, 1), pfg3)
    return dout.reshape(B_, 1, H_, W_)


# speedup vs baseline: 2.2023x; 1.0572x over previous
import functools

import jax
import jax.numpy as jnp
from jax import lax
from jax.experimental import pallas as pl
from jax.experimental.pallas import tpu as pltpu
from jax.experimental.pallas import tpu_sc as plsc

_SQRT1_2 = 0.7071067811865476
_NT = 512


def _gelu(x):
    return 0.5 * x * (1.0 + lax.erf(x * _SQRT1_2))


def _mm(x, w):
    m, s, c = x.shape
    y = lax.dot_general(x.reshape(m * s, c), w, (((1,), (1,)), ((), ())),
                        preferred_element_type=jnp.float32)
    return y.reshape(m, s, w.shape[0])


def _s1_body(if_ref, pf_ref, w1_ref, at_ref, bvt_ref):
    ifb = if_ref[...]
    x33 = jnp.concatenate([ifb, pf_ref[...]], axis=0)
    w1 = w1_ref[...]
    at_ref[...] = lax.dot_general(w1[:, 0:32], ifb, (((1,), (0,)), ((), ())),
                                  preferred_element_type=jnp.float32)
    bvt_ref[...] = lax.dot_general(x33, w1[:, 32:65], (((0,), (1,)), ((), ())),
                                   preferred_element_type=jnp.float32)


def _stage1(if2, pf2, w1, n):
    grid = n // _NT
    return pl.pallas_call(
        _s1_body,
        grid=(grid,),
        in_specs=[
            pl.BlockSpec((32, _NT), lambda g: (0, g)),
            pl.BlockSpec((1, _NT), lambda g: (0, g)),
            pl.BlockSpec((32, 67), lambda g: (0, 0)),
        ],
        out_specs=[
            pl.BlockSpec((32, _NT), lambda g: (0, g)),
            pl.BlockSpec((_NT, 32), lambda g: (g, 0)),
        ],
        out_shape=[jax.ShapeDtypeStruct((32, n), jnp.float32),
                   jax.ShapeDtypeStruct((n, 32), jnp.float32)],
    )(if2, pf2, w1)


def _sc_gather(bvt, idxf, pfflat, npts, n):
    NW = 32
    rw = npts // NW
    ch = 1008
    nch = rw // ch
    mesh = plsc.VectorSubcoreMesh(core_axis_name="c", subcore_axis_name="s")

    @functools.partial(
        pl.kernel,
        out_type=[jax.ShapeDtypeStruct((npts, 32), jnp.float32),
                  jax.ShapeDtypeStruct((npts,), jnp.float32)],
        mesh=mesh,
        scratch_types=[pltpu.VMEM((ch,), jnp.int32),
                       pltpu.VMEM((ch, 32), jnp.float32),
                       pltpu.VMEM((ch,), jnp.float32),
                       pltpu.SemaphoreType.DMA,
                       pltpu.SemaphoreType.DMA],
        compiler_params=pltpu.CompilerParams(use_tc_tiling_on_sc=False),
    )
    def k(table_hbm, idx_hbm, pf_hbm, g_out, pfg_out,
          idx_v, rows_v, pf_v, sem, sem2):
        wid = lax.axis_index("s") * 2 + lax.axis_index("c")
        base = wid * rw
        for kk in range(nch):
            cb = base + kk * ch
            pltpu.sync_copy(idx_hbm.at[pl.ds(cb, ch)], idx_v)
            cp1 = pltpu.async_copy(table_hbm.at[idx_v], rows_v, sem)
            cp2 = pltpu.async_copy(pf_hbm.at[idx_v], pf_v, sem2)
            cp1.wait()
            cp2.wait()
            pltpu.sync_copy(rows_v, g_out.at[pl.ds(cb, ch)])
            pltpu.sync_copy(pf_v, pfg_out.at[pl.ds(cb, ch)])

    return k(bvt, idxf, pfflat)


def _y1_cm(g_ref, at_ref, of_ref, w1o_ref):
    g3 = g_ref[...]
    gt = jnp.transpose(g3, (0, 2, 1))
    of = of_ref[...]
    w1o = w1o_ref[...]
    at = at_ref[...]
    num_ = g3.shape[0]
    rows = []
    for m in range(num_):
        cm = lax.dot_general(w1o, of[:, m, :], (((1,), (0,)), ((), ())),
                             preferred_element_type=jnp.float32)
        rows.append(gt[m] + at + cm)
    return jnp.stack(rows, axis=0)


def _mm_cm(x, w):
    return jnp.stack(
        [lax.dot_general(w, x[m], (((1,), (0,)), ((), ())),
                         preferred_element_type=jnp.float32)
         for m in range(x.shape[0])], axis=0)


def _acc_stats(out_ref, y):
    ps = jnp.stack([jnp.sum(y, axis=(0, 2)), jnp.sum(y * y, axis=(0, 2))])

    @pl.when(pl.program_id(0) == 0)
    def _():
        out_ref[...] = jnp.zeros_like(out_ref)

    out_ref[...] += ps


def _body0(g_ref, at_ref, of_ref, w1o_ref, y1_ref, st_ref):
    y1 = _y1_cm(g_ref, at_ref, of_ref, w1o_ref)
    y1_ref[...] = y1
    _acc_stats(st_ref, y1)


def _body1(y1_ref, sc1_ref, w2_ref, h_ref, st_ref):
    sc = sc1_ref[...]
    h = _gelu(y1_ref[...] * sc[0] + sc[1])
    h_ref[...] = h
    _acc_stats(st_ref, _mm_cm(h, w2_ref[...]))


def _body2(h_ref, w2f_ref, t2_ref, w3_ref, xf_ref, st_ref):
    h = h_ref[...]
    xf = _gelu(_mm_cm(h, w2f_ref[...]) + t2_ref[...][None])
    xf_ref[...] = xf
    _acc_stats(st_ref, _mm_cm(xf, w3_ref[...]))


def _body3(xf_ref, w3f_ref, t3_ref, w4_ref, st_ref):
    xf = xf_ref[...]
    l = _gelu(_mm_cm(xf, w3f_ref[...]) + t3_ref[...][None])
    _acc_stats(st_ref, _mm_cm(l, w4_ref[...]))


def _body4(xf_ref, w3f_ref, t3_ref, w4f_ref, t4_ref, wc_ref, bc_ref,
           pfg_ref, out_ref):
    xf = xf_ref[...]
    l = _gelu(_mm_cm(xf, w3f_ref[...]) + t3_ref[...][None])
    xl = _mm_cm(l, w4f_ref[...]) + t4_ref[...][None]
    xf2 = _gelu(xf + xl)
    wc = wc_ref[...]
    bcv = bc_ref[...]
    num = xf2.shape[0]
    feats = [lax.dot_general(wc, xf2[m], (((1,), (0,)), ((), ())),
                             preferred_element_type=jnp.float32) + bcv
             for m in range(num)]
    alpha = jnp.stack([f[0] for f in feats], axis=0)
    beta = jnp.stack([f[1] for f in feats], axis=0)
    om = jnp.stack([f[2] for f in feats], axis=0)
    om = om - jnp.max(om, axis=0, keepdims=True)
    e = jnp.exp(om)
    om = e / jnp.sum(e, axis=0, keepdims=True)
    d = jnp.sum(((alpha + 1.0) * pfg_ref[...] + beta) * om, axis=0)
    out_ref[...] = d[None]


_B_STAT = pl.BlockSpec((2, 32), lambda g: (0, 0))
_S_STAT = jax.ShapeDtypeStruct((2, 32), jnp.float32)


def _cm_spec(num):
    return pl.BlockSpec((num, 32, _NT), lambda g: (0, 0, g))


def _const(shape):
    nd = len(shape)
    return pl.BlockSpec(shape, lambda g: (0,) * nd)


def _scale_shift(stats, g, b, cnt):
    mean = stats[0] / cnt
    var = stats[1] / cnt - mean * mean
    s = g * lax.rsqrt(var + 1e-5)
    return s, b - mean * s


def kernel(If, Pf, Ofnum, args, W1, g1, b1, W2, g2, b2, W3, g3, b3,
           W4, g4, b4, Wc, bc):
    B_, Cfi_, H_, W_ = If.shape
    n = H_ * W_
    num = args.shape[-2]
    npts = num * n
    grid = (n // _NT,)

    if2 = If.reshape(Cfi_, n)
    pf2 = Pf.reshape(1, n)
    at, bvt = _stage1(if2, pf2, W1, n)

    idxf = args.reshape(npts)
    g_rows, pfg = _sc_gather(bvt, idxf, Pf.reshape(n), npts, n)
    g3_ = g_rows.reshape(num, n, 32)
    pfg3 = pfg.reshape(num, n)

    of3 = Ofnum.reshape(2, num, n)
    w1o = W1[:, 65:67]
    cnt = jnp.float32(npts)

    g_spec = pl.BlockSpec((num, _NT, 32), lambda g: (0, g, 0))
    at_spec = pl.BlockSpec((32, _NT), lambda g: (0, g))
    of_spec = pl.BlockSpec((2, num, _NT), lambda g: (0, 0, g))
    y1_ins = [g_spec, at_spec, of_spec, _const((32, 2))]
    cm_shape = jax.ShapeDtypeStruct((num, 32, n), jnp.float32)

    y1_, st1 = pl.pallas_call(
        _body0, grid=grid, in_specs=y1_ins,
        out_specs=[_cm_spec(num), _B_STAT],
        out_shape=[cm_shape, _S_STAT])(g3_, at, of3, w1o)
    s1, t1 = _scale_shift(st1, g1, b1, cnt)
    sc1 = jnp.stack([s1, t1]).reshape(2, 32, 1)

    h, st2 = pl.pallas_call(
        _body1, grid=grid,
        in_specs=[_cm_spec(num), _const((2, 32, 1)), _const((32, 32))],
        out_specs=[_cm_spec(num), _B_STAT],
        out_shape=[cm_shape, _S_STAT],
    )(y1_, sc1, W2)
    s2, t2 = _scale_shift(st2, g2, b2, cnt)

    xf, st3 = pl.pallas_call(
        _body2, grid=grid,
        in_specs=[_cm_spec(num), _const((32, 32)), _const((32, 1)),
                  _const((32, 32))],
        out_specs=[_cm_spec(num), _B_STAT],
        out_shape=[jax.ShapeDtypeStruct((num, 32, n), jnp.float32), _S_STAT],
    )(h, W2 * s2[:, None], t2.reshape(32, 1), W3)
    s3, t3 = _scale_shift(st3, g3, b3, cnt)

    st4 = pl.pallas_call(
        _body3, grid=grid,
        in_specs=[_cm_spec(num), _const((32, 32)), _const((32, 1)),
                  _const((32, 32))],
        out_specs=_B_STAT, out_shape=_S_STAT,
    )(xf, W3 * s3[:, None], t3.reshape(32, 1), W4)
    s4, t4 = _scale_shift(st4, g4, b4, cnt)

    dout = pl.pallas_call(
        _body4, grid=grid,
        in_specs=[_cm_spec(num), _const((32, 32)), _const((32, 1)),
                  _const((32, 32)), _const((32, 1)), _const((3, 32)),
                  _const((3, 1)),
                  pl.BlockSpec((num, _NT), lambda g: (0, g))],
        out_specs=pl.BlockSpec((1, _NT), lambda g: (0, g)),
        out_shape=jax.ShapeDtypeStruct((1, n), jnp.float32),
    )(xf, W3 * s3[:, None], t3.reshape(32, 1), W4 * s4[:, None],
      t4.reshape(32, 1), Wc, bc.reshape(3, 1), pfg3)
    return dout.reshape(B_, 1, H_, W_)


# speedup vs baseline: 2.6670x; 1.2110x over previous
import functools

import jax
import jax.numpy as jnp
from jax import lax
from jax.experimental import pallas as pl
from jax.experimental.pallas import tpu as pltpu
from jax.experimental.pallas import tpu_sc as plsc

_SQRT1_2 = 0.7071067811865476
_NT = 512


def _gelu(x):
    return 0.5 * x * (1.0 + lax.erf(x * _SQRT1_2))


def _mm(x, w):
    m, s, c = x.shape
    y = lax.dot_general(x.reshape(m * s, c), w, (((1,), (1,)), ((), ())),
                        preferred_element_type=jnp.float32)
    return y.reshape(m, s, w.shape[0])


def _s1_body(if_ref, pf_ref, w1_ref, at_ref, bvt_ref):
    ifb = if_ref[...]
    x33 = jnp.concatenate([ifb, pf_ref[...]], axis=0)
    w1 = w1_ref[...]
    at_ref[...] = lax.dot_general(w1[:, 0:32], ifb, (((1,), (0,)), ((), ())),
                                  preferred_element_type=jnp.float32)
    bvt_ref[...] = lax.dot_general(x33, w1[:, 32:65], (((0,), (1,)), ((), ())),
                                   preferred_element_type=jnp.float32)


def _stage1(if2, pf2, w1, n):
    grid = n // _NT
    return pl.pallas_call(
        _s1_body,
        grid=(grid,),
        in_specs=[
            pl.BlockSpec((32, _NT), lambda g: (0, g)),
            pl.BlockSpec((1, _NT), lambda g: (0, g)),
            pl.BlockSpec((32, 67), lambda g: (0, 0)),
        ],
        out_specs=[
            pl.BlockSpec((32, _NT), lambda g: (0, g)),
            pl.BlockSpec((_NT, 32), lambda g: (g, 0)),
        ],
        out_shape=[jax.ShapeDtypeStruct((32, n), jnp.float32),
                   jax.ShapeDtypeStruct((n, 32), jnp.float32)],
    )(if2, pf2, w1)


def _sc_gather(bvt, idxf, pfflat, npts, n):
    NW = 32
    rw = npts // NW
    ch = 1008
    nch = rw // ch
    mesh = plsc.VectorSubcoreMesh(core_axis_name="c", subcore_axis_name="s")

    @functools.partial(
        pl.kernel,
        out_type=[jax.ShapeDtypeStruct((npts, 128), jnp.float32),
                  jax.ShapeDtypeStruct((npts,), jnp.float32)],
        mesh=mesh,
        scratch_types=[pltpu.VMEM((ch,), jnp.int32),
                       pltpu.VMEM((ch, 32), jnp.float32),
                       pltpu.VMEM((ch,), jnp.float32),
                       pltpu.SemaphoreType.DMA,
                       pltpu.SemaphoreType.DMA],
        compiler_params=pltpu.CompilerParams(use_tc_tiling_on_sc=False),
    )
    def k(table_hbm, idx_hbm, pf_hbm, g_out, pfg_out,
          idx_v, rows_v, pf_v, sem, sem2):
        wid = lax.axis_index("s") * 2 + lax.axis_index("c")
        base = wid * rw
        for kk in range(nch):
            cb = base + kk * ch
            pltpu.sync_copy(idx_hbm.at[pl.ds(cb, ch)], idx_v)
            cp1 = pltpu.async_copy(table_hbm.at[idx_v], rows_v, sem)
            cp2 = pltpu.async_copy(pf_hbm.at[idx_v], pf_v, sem2)
            cp1.wait()
            cp2.wait()
            pltpu.sync_copy(rows_v, g_out.at[pl.ds(cb, ch), pl.ds(0, 32)])
            pltpu.sync_copy(pf_v, pfg_out.at[pl.ds(cb, ch)])

    return k(bvt, idxf, pfflat)


def _y1_cm(g_ref, at_ref, of_ref, w1o_ref):
    g3 = g_ref[...][:, :, 0:32]
    gt = jnp.transpose(g3, (0, 2, 1))
    of = of_ref[...]
    w1o = w1o_ref[...]
    at = at_ref[...]
    num_ = g3.shape[0]
    rows = []
    for m in range(num_):
        cm = lax.dot_general(w1o, of[:, m, :], (((1,), (0,)), ((), ())),
                             preferred_element_type=jnp.float32)
        rows.append(gt[m] + at + cm)
    return jnp.stack(rows, axis=0)


def _mm_cm(x, w):
    return jnp.stack(
        [lax.dot_general(w, x[m], (((1,), (0,)), ((), ())),
                         preferred_element_type=jnp.float32)
         for m in range(x.shape[0])], axis=0)


def _acc_stats(out_ref, y):
    ps = jnp.stack([jnp.sum(y, axis=(0, 2)), jnp.sum(y * y, axis=(0, 2))])

    @pl.when(pl.program_id(0) == 0)
    def _():
        out_ref[...] = jnp.zeros_like(out_ref)

    out_ref[...] += ps


def _body0(g_ref, at_ref, of_ref, w1o_ref, y1_ref, st_ref):
    y1 = _y1_cm(g_ref, at_ref, of_ref, w1o_ref)
    y1_ref[...] = y1
    _acc_stats(st_ref, y1)


def _body1(y1_ref, sc1_ref, w2_ref, h_ref, st_ref):
    sc = sc1_ref[...]
    h = _gelu(y1_ref[...] * sc[0] + sc[1])
    h_ref[...] = h
    _acc_stats(st_ref, _mm_cm(h, w2_ref[...]))


def _body2(h_ref, w2f_ref, t2_ref, w3_ref, xf_ref, st_ref):
    h = h_ref[...]
    xf = _gelu(_mm_cm(h, w2f_ref[...]) + t2_ref[...][None])
    xf_ref[...] = xf
    _acc_stats(st_ref, _mm_cm(xf, w3_ref[...]))


def _body3(xf_ref, w3f_ref, t3_ref, w4_ref, st_ref):
    xf = xf_ref[...]
    l = _gelu(_mm_cm(xf, w3f_ref[...]) + t3_ref[...][None])
    _acc_stats(st_ref, _mm_cm(l, w4_ref[...]))


def _body4(xf_ref, w3f_ref, t3_ref, w4f_ref, t4_ref, wc_ref, bc_ref,
           pfg_ref, out_ref):
    xf = xf_ref[...]
    l = _gelu(_mm_cm(xf, w3f_ref[...]) + t3_ref[...][None])
    xl = _mm_cm(l, w4f_ref[...]) + t4_ref[...][None]
    xf2 = _gelu(xf + xl)
    wc = wc_ref[...]
    bcv = bc_ref[...]
    num = xf2.shape[0]
    feats = [lax.dot_general(wc, xf2[m], (((1,), (0,)), ((), ())),
                             preferred_element_type=jnp.float32) + bcv
             for m in range(num)]
    alpha = jnp.stack([f[0] for f in feats], axis=0)
    beta = jnp.stack([f[1] for f in feats], axis=0)
    om = jnp.stack([f[2] for f in feats], axis=0)
    om = om - jnp.max(om, axis=0, keepdims=True)
    e = jnp.exp(om)
    om = e / jnp.sum(e, axis=0, keepdims=True)
    d = jnp.sum(((alpha + 1.0) * pfg_ref[...] + beta) * om, axis=0)
    out_ref[...] = d[None]


_B_STAT = pl.BlockSpec((2, 32), lambda g: (0, 0))
_S_STAT = jax.ShapeDtypeStruct((2, 32), jnp.float32)


def _cm_spec(num):
    return pl.BlockSpec((num, 32, _NT), lambda g: (0, 0, g))


def _const(shape):
    nd = len(shape)
    return pl.BlockSpec(shape, lambda g: (0,) * nd)


def _scale_shift(stats, g, b, cnt):
    mean = stats[0] / cnt
    var = stats[1] / cnt - mean * mean
    s = g * lax.rsqrt(var + 1e-5)
    return s, b - mean * s


def kernel(If, Pf, Ofnum, args, W1, g1, b1, W2, g2, b2, W3, g3, b3,
           W4, g4, b4, Wc, bc):
    B_, Cfi_, H_, W_ = If.shape
    n = H_ * W_
    num = args.shape[-2]
    npts = num * n
    grid = (n // _NT,)

    if2 = If.reshape(Cfi_, n)
    pf2 = Pf.reshape(1, n)
    at, bvt = _stage1(if2, pf2, W1, n)

    idxf = args.reshape(npts)
    g_rows, pfg = _sc_gather(bvt, idxf, Pf.reshape(n), npts, n)
    g3_ = g_rows.reshape(num, n, 128)
    pfg3 = pfg.reshape(num, n)

    of3 = Ofnum.reshape(2, num, n)
    w1o = W1[:, 65:67]
    cnt = jnp.float32(npts)

    g_spec = pl.BlockSpec((num, _NT, 128), lambda g: (0, g, 0))
    at_spec = pl.BlockSpec((32, _NT), lambda g: (0, g))
    of_spec = pl.BlockSpec((2, num, _NT), lambda g: (0, 0, g))
    y1_ins = [g_spec, at_spec, of_spec, _const((32, 2))]
    cm_shape = jax.ShapeDtypeStruct((num, 32, n), jnp.float32)

    y1_, st1 = pl.pallas_call(
        _body0, grid=grid, in_specs=y1_ins,
        out_specs=[_cm_spec(num), _B_STAT],
        out_shape=[cm_shape, _S_STAT])(g3_, at, of3, w1o)
    s1, t1 = _scale_shift(st1, g1, b1, cnt)
    sc1 = jnp.stack([s1, t1]).reshape(2, 32, 1)

    h, st2 = pl.pallas_call(
        _body1, grid=grid,
        in_specs=[_cm_spec(num), _const((2, 32, 1)), _const((32, 32))],
        out_specs=[_cm_spec(num), _B_STAT],
        out_shape=[cm_shape, _S_STAT],
    )(y1_, sc1, W2)
    s2, t2 = _scale_shift(st2, g2, b2, cnt)

    xf, st3 = pl.pallas_call(
        _body2, grid=grid,
        in_specs=[_cm_spec(num), _const((32, 32)), _const((32, 1)),
                  _const((32, 32))],
        out_specs=[_cm_spec(num), _B_STAT],
        out_shape=[jax.ShapeDtypeStruct((num, 32, n), jnp.float32), _S_STAT],
    )(h, W2 * s2[:, None], t2.reshape(32, 1), W3)
    s3, t3 = _scale_shift(st3, g3, b3, cnt)

    st4 = pl.pallas_call(
        _body3, grid=grid,
        in_specs=[_cm_spec(num), _const((32, 32)), _const((32, 1)),
                  _const((32, 32))],
        out_specs=_B_STAT, out_shape=_S_STAT,
    )(xf, W3 * s3[:, None], t3.reshape(32, 1), W4)
    s4, t4 = _scale_shift(st4, g4, b4, cnt)

    dout = pl.pallas_call(
        _body4, grid=grid,
        in_specs=[_cm_spec(num), _const((32, 32)), _const((32, 1)),
                  _const((32, 32)), _const((32, 1)), _const((3, 32)),
                  _const((3, 1)),
                  pl.BlockSpec((num, _NT), lambda g: (0, g))],
        out_specs=pl.BlockSpec((1, _NT), lambda g: (0, g)),
        out_shape=jax.ShapeDtypeStruct((1, n), jnp.float32),
    )(xf, W3 * s3[:, None], t3.reshape(32, 1), W4 * s4[:, None],
      t4.reshape(32, 1), Wc, bc.reshape(3, 1), pfg3)
    return dout.reshape(B_, 1, H_, W_)


# speedup vs baseline: 2.7695x; 1.0384x over previous
import functools

import jax
import jax.numpy as jnp
from jax import lax
from jax.experimental import pallas as pl
from jax.experimental.pallas import tpu as pltpu
from jax.experimental.pallas import tpu_sc as plsc

_SQRT1_2 = 0.7071067811865476
_NT = 512


def _gelu(x):
    return 0.5 * x * (1.0 + lax.erf(x * _SQRT1_2))


def _mm(x, w):
    m, s, c = x.shape
    y = lax.dot_general(x.reshape(m * s, c), w, (((1,), (1,)), ((), ())),
                        preferred_element_type=jnp.float32)
    return y.reshape(m, s, w.shape[0])


def _s1_body(if_ref, pf_ref, w1_ref, at_ref, bvt_ref):
    ifb = if_ref[...]
    x33 = jnp.concatenate([ifb, pf_ref[...]], axis=0)
    w1 = w1_ref[...]
    at_ref[...] = lax.dot_general(w1[:, 0:32], ifb, (((1,), (0,)), ((), ())),
                                  preferred_element_type=jnp.float32)
    bvt_ref[...] = lax.dot_general(x33, w1[:, 32:65], (((0,), (1,)), ((), ())),
                                   preferred_element_type=jnp.float32)


def _stage1(if2, pf2, w1, n):
    grid = n // _NT
    return pl.pallas_call(
        _s1_body,
        grid=(grid,),
        in_specs=[
            pl.BlockSpec((32, _NT), lambda g: (0, g)),
            pl.BlockSpec((1, _NT), lambda g: (0, g)),
            pl.BlockSpec((32, 67), lambda g: (0, 0)),
        ],
        out_specs=[
            pl.BlockSpec((32, _NT), lambda g: (0, g)),
            pl.BlockSpec((_NT, 32), lambda g: (g, 0)),
        ],
        out_shape=[jax.ShapeDtypeStruct((32, n), jnp.float32),
                   jax.ShapeDtypeStruct((n, 32), jnp.float32)],
    )(if2, pf2, w1)


def _sc_gather(bvt, idxf, pfflat, npts, n):
    NW = 32
    rw = npts // NW
    ch = 1008
    nch = rw // ch
    mesh = plsc.VectorSubcoreMesh(core_axis_name="c", subcore_axis_name="s")

    @functools.partial(
        pl.kernel,
        out_type=[jax.ShapeDtypeStruct((npts, 128), jnp.float32),
                  jax.ShapeDtypeStruct((npts,), jnp.float32)],
        mesh=mesh,
        scratch_types=[pltpu.VMEM((ch,), jnp.int32),
                       pltpu.VMEM((ch, 32), jnp.float32),
                       pltpu.VMEM((ch,), jnp.float32),
                       pltpu.SemaphoreType.DMA,
                       pltpu.SemaphoreType.DMA],
        compiler_params=pltpu.CompilerParams(use_tc_tiling_on_sc=False),
    )
    def k(table_hbm, idx_hbm, pf_hbm, g_out, pfg_out,
          idx_v, rows_v, pf_v, sem, sem2):
        wid = lax.axis_index("s") * 2 + lax.axis_index("c")
        base = wid * rw
        for kk in range(nch):
            cb = base + kk * ch
            pltpu.sync_copy(idx_hbm.at[pl.ds(cb, ch)], idx_v)
            cp1 = pltpu.async_copy(table_hbm.at[idx_v], rows_v, sem)
            cp2 = pltpu.async_copy(pf_hbm.at[idx_v], pf_v, sem2)
            cp1.wait()
            cp2.wait()
            pltpu.sync_copy(rows_v, g_out.at[pl.ds(cb, ch), pl.ds(0, 32)])
            pltpu.sync_copy(pf_v, pfg_out.at[pl.ds(cb, ch)])

    return k(bvt, idxf, pfflat)


def _y1_cm(g_ref, at_ref, of_ref, w1o_ref):
    g3 = g_ref[...][:, :, 0:32]
    gt = jnp.transpose(g3, (0, 2, 1))
    of = of_ref[...]
    w1o = w1o_ref[...]
    at = at_ref[...]
    num_ = g3.shape[0]
    rows = []
    for m in range(num_):
        cm = lax.dot_general(w1o, of[:, m, :], (((1,), (0,)), ((), ())),
                             preferred_element_type=jnp.float32)
        rows.append(gt[m] + at + cm)
    return jnp.stack(rows, axis=0)


def _mm_cm(x, w):
    return jnp.stack(
        [lax.dot_general(w, x[m], (((1,), (0,)), ((), ())),
                         preferred_element_type=jnp.float32)
         for m in range(x.shape[0])], axis=0)


def _acc_stats(out_ref, y):
    ps = jnp.stack([jnp.sum(y, axis=(0, 2)), jnp.sum(y * y, axis=(0, 2))])

    @pl.when(pl.program_id(0) == 0)
    def _():
        out_ref[...] = jnp.zeros_like(out_ref)

    out_ref[...] += ps


def _body0(g_ref, at_ref, of_ref, w1o_ref, y1_ref, st_ref):
    y1 = _y1_cm(g_ref, at_ref, of_ref, w1o_ref)
    y1_ref[...] = y1
    _acc_stats(st_ref, y1)


def _body1(y1_ref, sc1_ref, w2_ref, h_ref, st_ref):
    sc = sc1_ref[...]
    h = _gelu(y1_ref[...] * sc[0] + sc[1])
    h_ref[...] = h.astype(jnp.bfloat16)
    _acc_stats(st_ref, _mm_cm(h, w2_ref[...]))


def _body2(h_ref, w2f_ref, t2_ref, w3_ref, xf_ref, st_ref):
    h = h_ref[...].astype(jnp.float32)
    xf = _gelu(_mm_cm(h, w2f_ref[...]) + t2_ref[...][None])
    xf_ref[...] = xf.astype(jnp.bfloat16)
    _acc_stats(st_ref, _mm_cm(xf, w3_ref[...]))


def _body3(xf_ref, w3f_ref, t3_ref, w4_ref, st_ref):
    xf = xf_ref[...].astype(jnp.float32)
    l = _gelu(_mm_cm(xf, w3f_ref[...]) + t3_ref[...][None])
    _acc_stats(st_ref, _mm_cm(l, w4_ref[...]))


def _body4(xf_ref, w3f_ref, t3_ref, w4f_ref, t4_ref, wc_ref, bc_ref,
           pfg_ref, out_ref):
    xf = xf_ref[...].astype(jnp.float32)
    l = _gelu(_mm_cm(xf, w3f_ref[...]) + t3_ref[...][None])
    xl = _mm_cm(l, w4f_ref[...]) + t4_ref[...][None]
    xf2 = _gelu(xf + xl)
    wc = wc_ref[...]
    bcv = bc_ref[...]
    num = xf2.shape[0]
    feats = [lax.dot_general(wc, xf2[m], (((1,), (0,)), ((), ())),
                             preferred_element_type=jnp.float32) + bcv
             for m in range(num)]
    alpha = jnp.stack([f[0] for f in feats], axis=0)
    beta = jnp.stack([f[1] for f in feats], axis=0)
    om = jnp.stack([f[2] for f in feats], axis=0)
    om = om - jnp.max(om, axis=0, keepdims=True)
    e = jnp.exp(om)
    om = e / jnp.sum(e, axis=0, keepdims=True)
    d = jnp.sum(((alpha + 1.0) * pfg_ref[...] + beta) * om, axis=0)
    out_ref[...] = d[None]


_B_STAT = pl.BlockSpec((2, 32), lambda g: (0, 0))
_S_STAT = jax.ShapeDtypeStruct((2, 32), jnp.float32)


def _cm_spec(num):
    return pl.BlockSpec((num, 32, _NT), lambda g: (0, 0, g))


def _const(shape):
    nd = len(shape)
    return pl.BlockSpec(shape, lambda g: (0,) * nd)


def _scale_shift(stats, g, b, cnt):
    mean = stats[0] / cnt
    var = stats[1] / cnt - mean * mean
    s = g * lax.rsqrt(var + 1e-5)
    return s, b - mean * s


def kernel(If, Pf, Ofnum, args, W1, g1, b1, W2, g2, b2, W3, g3, b3,
           W4, g4, b4, Wc, bc):
    B_, Cfi_, H_, W_ = If.shape
    n = H_ * W_
    num = args.shape[-2]
    npts = num * n
    grid = (n // _NT,)

    if2 = If.reshape(Cfi_, n)
    pf2 = Pf.reshape(1, n)
    at, bvt = _stage1(if2, pf2, W1, n)

    idxf = args.reshape(npts)
    g_rows, pfg = _sc_gather(bvt, idxf, Pf.reshape(n), npts, n)
    g3_ = g_rows.reshape(num, n, 128)
    pfg3 = pfg.reshape(num, n)

    of3 = Ofnum.reshape(2, num, n)
    w1o = W1[:, 65:67]
    cnt = jnp.float32(npts)

    g_spec = pl.BlockSpec((num, _NT, 128), lambda g: (0, g, 0))
    at_spec = pl.BlockSpec((32, _NT), lambda g: (0, g))
    of_spec = pl.BlockSpec((2, num, _NT), lambda g: (0, 0, g))
    y1_ins = [g_spec, at_spec, of_spec, _const((32, 2))]
    cm_shape = jax.ShapeDtypeStruct((num, 32, n), jnp.float32)
    cm_bf16 = jax.ShapeDtypeStruct((num, 32, n), jnp.bfloat16)

    y1_, st1 = pl.pallas_call(
        _body0, grid=grid, in_specs=y1_ins,
        out_specs=[_cm_spec(num), _B_STAT],
        out_shape=[cm_shape, _S_STAT])(g3_, at, of3, w1o)
    s1, t1 = _scale_shift(st1, g1, b1, cnt)
    sc1 = jnp.stack([s1, t1]).reshape(2, 32, 1)

    h, st2 = pl.pallas_call(
        _body1, grid=grid,
        in_specs=[_cm_spec(num), _const((2, 32, 1)), _const((32, 32))],
        out_specs=[_cm_spec(num), _B_STAT],
        out_shape=[cm_bf16, _S_STAT],
    )(y1_, sc1, W2)
    s2, t2 = _scale_shift(st2, g2, b2, cnt)

    xf, st3 = pl.pallas_call(
        _body2, grid=grid,
        in_specs=[_cm_spec(num), _const((32, 32)), _const((32, 1)),
                  _const((32, 32))],
        out_specs=[_cm_spec(num), _B_STAT],
        out_shape=[cm_bf16, _S_STAT],
    )(h, W2 * s2[:, None], t2.reshape(32, 1), W3)
    s3, t3 = _scale_shift(st3, g3, b3, cnt)

    st4 = pl.pallas_call(
        _body3, grid=grid,
        in_specs=[_cm_spec(num), _const((32, 32)), _const((32, 1)),
                  _const((32, 32))],
        out_specs=_B_STAT, out_shape=_S_STAT,
    )(xf, W3 * s3[:, None], t3.reshape(32, 1), W4)
    s4, t4 = _scale_shift(st4, g4, b4, cnt)

    dout = pl.pallas_call(
        _body4, grid=grid,
        in_specs=[_cm_spec(num), _const((32, 32)), _const((32, 1)),
                  _const((32, 32)), _const((32, 1)), _const((3, 32)),
                  _const((3, 1)),
                  pl.BlockSpec((num, _NT), lambda g: (0, g))],
        out_specs=pl.BlockSpec((1, _NT), lambda g: (0, g)),
        out_shape=jax.ShapeDtypeStruct((1, n), jnp.float32),
    )(xf, W3 * s3[:, None], t3.reshape(32, 1), W4 * s4[:, None],
      t4.reshape(32, 1), Wc, bc.reshape(3, 1), pfg3)
    return dout.reshape(B_, 1, H_, W_)


# speedup vs baseline: 2.8293x; 1.0216x over previous
import functools

import jax
import jax.numpy as jnp
from jax import lax
from jax.experimental import pallas as pl
from jax.experimental.pallas import tpu as pltpu
from jax.experimental.pallas import tpu_sc as plsc

_SQRT1_2 = 0.7071067811865476
_NT = 512


def _gelu(x):
    return 0.5 * x * (1.0 + lax.erf(x * _SQRT1_2))


def _mm(x, w):
    m, s, c = x.shape
    y = lax.dot_general(x.reshape(m * s, c), w, (((1,), (1,)), ((), ())),
                        preferred_element_type=jnp.float32)
    return y.reshape(m, s, w.shape[0])


def _s1_body(if_ref, pf_ref, w1_ref, at_ref, bvt_ref):
    ifb = if_ref[...]
    x33 = jnp.concatenate([ifb, pf_ref[...]], axis=0)
    w1 = w1_ref[...]
    at_ref[...] = lax.dot_general(w1[:, 0:32], ifb, (((1,), (0,)), ((), ())),
                                  preferred_element_type=jnp.float32)
    bvt_ref[...] = lax.dot_general(x33, w1[:, 32:65], (((0,), (1,)), ((), ())),
                                   preferred_element_type=jnp.float32)


def _stage1(if2, pf2, w1, n):
    grid = n // _NT
    return pl.pallas_call(
        _s1_body,
        grid=(grid,),
        in_specs=[
            pl.BlockSpec((32, _NT), lambda g: (0, g)),
            pl.BlockSpec((1, _NT), lambda g: (0, g)),
            pl.BlockSpec((32, 67), lambda g: (0, 0)),
        ],
        out_specs=[
            pl.BlockSpec((32, _NT), lambda g: (0, g)),
            pl.BlockSpec((_NT, 32), lambda g: (g, 0)),
        ],
        out_shape=[jax.ShapeDtypeStruct((32, n), jnp.float32),
                   jax.ShapeDtypeStruct((n, 32), jnp.float32)],
    )(if2, pf2, w1)


def _sc_gather(bvt, idxf, pfflat, npts, n):
    NW = 32
    rw = npts // NW
    ch = 1008
    nch = rw // ch
    mesh = plsc.VectorSubcoreMesh(core_axis_name="c", subcore_axis_name="s")

    @functools.partial(
        pl.kernel,
        out_type=[jax.ShapeDtypeStruct((npts, 128), jnp.float32),
                  jax.ShapeDtypeStruct((npts,), jnp.float32)],
        mesh=mesh,
        scratch_types=[pltpu.VMEM((ch,), jnp.int32),
                       pltpu.VMEM((ch, 32), jnp.float32),
                       pltpu.VMEM((ch,), jnp.float32),
                       pltpu.SemaphoreType.DMA,
                       pltpu.SemaphoreType.DMA],
        compiler_params=pltpu.CompilerParams(use_tc_tiling_on_sc=False),
    )
    def k(table_hbm, idx_hbm, pf_hbm, g_out, pfg_out,
          idx_v, rows_v, pf_v, sem, sem2):
        wid = lax.axis_index("s") * 2 + lax.axis_index("c")
        base = wid * rw
        for kk in range(nch):
            cb = base + kk * ch
            pltpu.sync_copy(idx_hbm.at[pl.ds(cb, ch)], idx_v)
            cp1 = pltpu.async_copy(table_hbm.at[idx_v], rows_v, sem)
            cp2 = pltpu.async_copy(pf_hbm.at[idx_v], pf_v, sem2)
            cp1.wait()
            cp2.wait()
            pltpu.sync_copy(rows_v, g_out.at[pl.ds(cb, ch), pl.ds(0, 32)])
            pltpu.sync_copy(pf_v, pfg_out.at[pl.ds(cb, ch)])

    return k(bvt, idxf, pfflat)


def _y1_cm(g_ref, at_ref, of_ref, w1o_ref):
    g3 = g_ref[...][:, :, 0:32]
    gt = jnp.transpose(g3, (0, 2, 1))
    of = of_ref[...]
    w1o = w1o_ref[...]
    at = at_ref[...]
    num_ = g3.shape[0]
    rows = []
    for m in range(num_):
        cm = lax.dot_general(w1o, of[:, m, :], (((1,), (0,)), ((), ())),
                             preferred_element_type=jnp.float32)
        rows.append(gt[m] + at + cm)
    return jnp.stack(rows, axis=0)


def _mm_cm(x, w):
    return jnp.stack(
        [lax.dot_general(w, x[m], (((1,), (0,)), ((), ())),
                         preferred_element_type=jnp.float32)
         for m in range(x.shape[0])], axis=0)


def _acc_stats(out_ref, y):
    ps = jnp.stack([jnp.sum(y, axis=(0, 2)), jnp.sum(y * y, axis=(0, 2))])

    @pl.when(pl.program_id(0) == 0)
    def _():
        out_ref[...] = jnp.zeros_like(out_ref)

    out_ref[...] += ps


def _body0(g_ref, at_ref, of_ref, w1o_ref, y1_ref, st_ref):
    y1 = _y1_cm(g_ref, at_ref, of_ref, w1o_ref)
    y1_ref[...] = y1.astype(jnp.bfloat16)
    _acc_stats(st_ref, y1)


def _body1(y1_ref, sc1_ref, w2_ref, h_ref, st_ref):
    sc = sc1_ref[...]
    h = _gelu(y1_ref[...].astype(jnp.float32) * sc[0] + sc[1])
    h_ref[...] = h.astype(jnp.bfloat16)
    _acc_stats(st_ref, _mm_cm(h, w2_ref[...]))


def _body2(h_ref, w2f_ref, t2_ref, w3_ref, xf_ref, st_ref):
    h = h_ref[...].astype(jnp.float32)
    xf = _gelu(_mm_cm(h, w2f_ref[...]) + t2_ref[...][None])
    xf_ref[...] = xf.astype(jnp.bfloat16)
    _acc_stats(st_ref, _mm_cm(xf, w3_ref[...]))


def _body3(xf_ref, w3f_ref, t3_ref, w4_ref, st_ref):
    xf = xf_ref[...].astype(jnp.float32)
    l = _gelu(_mm_cm(xf, w3f_ref[...]) + t3_ref[...][None])
    _acc_stats(st_ref, _mm_cm(l, w4_ref[...]))


def _body4(xf_ref, w3f_ref, t3_ref, w4f_ref, t4_ref, wc_ref, bc_ref,
           pfg_ref, out_ref):
    xf = xf_ref[...].astype(jnp.float32)
    l = _gelu(_mm_cm(xf, w3f_ref[...]) + t3_ref[...][None])
    xl = _mm_cm(l, w4f_ref[...]) + t4_ref[...][None]
    xf2 = _gelu(xf + xl)
    wc = wc_ref[...]
    bcv = bc_ref[...]
    num = xf2.shape[0]
    feats = [lax.dot_general(wc, xf2[m], (((1,), (0,)), ((), ())),
                             preferred_element_type=jnp.float32) + bcv
             for m in range(num)]
    alpha = jnp.stack([f[0] for f in feats], axis=0)
    beta = jnp.stack([f[1] for f in feats], axis=0)
    om = jnp.stack([f[2] for f in feats], axis=0)
    om = om - jnp.max(om, axis=0, keepdims=True)
    e = jnp.exp(om)
    om = e / jnp.sum(e, axis=0, keepdims=True)
    d = jnp.sum(((alpha + 1.0) * pfg_ref[...] + beta) * om, axis=0)
    out_ref[...] = d[None]


_B_STAT = pl.BlockSpec((2, 32), lambda g: (0, 0))
_S_STAT = jax.ShapeDtypeStruct((2, 32), jnp.float32)


def _cm_spec(num):
    return pl.BlockSpec((num, 32, _NT), lambda g: (0, 0, g))


def _const(shape):
    nd = len(shape)
    return pl.BlockSpec(shape, lambda g: (0,) * nd)


def _scale_shift(stats, g, b, cnt):
    mean = stats[0] / cnt
    var = stats[1] / cnt - mean * mean
    s = g * lax.rsqrt(var + 1e-5)
    return s, b - mean * s


def kernel(If, Pf, Ofnum, args, W1, g1, b1, W2, g2, b2, W3, g3, b3,
           W4, g4, b4, Wc, bc):
    B_, Cfi_, H_, W_ = If.shape
    n = H_ * W_
    num = args.shape[-2]
    npts = num * n
    grid = (n // _NT,)

    if2 = If.reshape(Cfi_, n)
    pf2 = Pf.reshape(1, n)
    at, bvt = _stage1(if2, pf2, W1, n)

    idxf = args.reshape(npts)
    g_rows, pfg = _sc_gather(bvt, idxf, Pf.reshape(n), npts, n)
    g3_ = g_rows.reshape(num, n, 128)
    pfg3 = pfg.reshape(num, n)

    of3 = Ofnum.reshape(2, num, n)
    w1o = W1[:, 65:67]
    cnt = jnp.float32(npts)

    g_spec = pl.BlockSpec((num, _NT, 128), lambda g: (0, g, 0))
    at_spec = pl.BlockSpec((32, _NT), lambda g: (0, g))
    of_spec = pl.BlockSpec((2, num, _NT), lambda g: (0, 0, g))
    y1_ins = [g_spec, at_spec, of_spec, _const((32, 2))]
    cm_shape = jax.ShapeDtypeStruct((num, 32, n), jnp.float32)
    cm_bf16 = jax.ShapeDtypeStruct((num, 32, n), jnp.bfloat16)

    y1_, st1 = pl.pallas_call(
        _body0, grid=grid, in_specs=y1_ins,
        out_specs=[_cm_spec(num), _B_STAT],
        out_shape=[cm_bf16, _S_STAT])(g3_, at, of3, w1o)
    s1, t1 = _scale_shift(st1, g1, b1, cnt)
    sc1 = jnp.stack([s1, t1]).reshape(2, 32, 1)

    h, st2 = pl.pallas_call(
        _body1, grid=grid,
        in_specs=[_cm_spec(num), _const((2, 32, 1)), _const((32, 32))],
        out_specs=[_cm_spec(num), _B_STAT],
        out_shape=[cm_bf16, _S_STAT],
    )(y1_, sc1, W2)
    s2, t2 = _scale_shift(st2, g2, b2, cnt)

    xf, st3 = pl.pallas_call(
        _body2, grid=grid,
        in_specs=[_cm_spec(num), _const((32, 32)), _const((32, 1)),
                  _const((32, 32))],
        out_specs=[_cm_spec(num), _B_STAT],
        out_shape=[cm_bf16, _S_STAT],
    )(h, W2 * s2[:, None], t2.reshape(32, 1), W3)
    s3, t3 = _scale_shift(st3, g3, b3, cnt)

    st4 = pl.pallas_call(
        _body3, grid=grid,
        in_specs=[_cm_spec(num), _const((32, 32)), _const((32, 1)),
                  _const((32, 32))],
        out_specs=_B_STAT, out_shape=_S_STAT,
    )(xf, W3 * s3[:, None], t3.reshape(32, 1), W4)
    s4, t4 = _scale_shift(st4, g4, b4, cnt)

    dout = pl.pallas_call(
        _body4, grid=grid,
        in_specs=[_cm_spec(num), _const((32, 32)), _const((32, 1)),
                  _const((32, 32)), _const((32, 1)), _const((3, 32)),
                  _const((3, 1)),
                  pl.BlockSpec((num, _NT), lambda g: (0, g))],
        out_specs=pl.BlockSpec((1, _NT), lambda g: (0, g)),
        out_shape=jax.ShapeDtypeStruct((1, n), jnp.float32),
    )(xf, W3 * s3[:, None], t3.reshape(32, 1), W4 * s4[:, None],
      t4.reshape(32, 1), Wc, bc.reshape(3, 1), pfg3)
    return dout.reshape(B_, 1, H_, W_)
